# baseline (speedup 1.0000x reference)
import functools

import jax
import jax.numpy as jnp
from jax import lax
from jax.experimental import pallas as pl
from jax.experimental.pallas import tpu as pltpu
from jax.experimental.pallas import tpu_sc as plsc

N = 10000
NE = 128
HID = 8
NHEADS = 8
Q = 7
ALPHA = 0.2

NPAD = 10240
NW = 32
NNZ = 100000
NNZ_CH = 25
NNZ_PAD = NW * NNZ_CH * 128
E = N * 16 + N
E_CH = 42
E_PAD = NW * E_CH * 128
XWORDS = NPAD * NE

_mesh = plsc.VectorSubcoreMesh(core_axis_name="c", subcore_axis_name="s")


def _densify_body(rows_hbm, cols_hbm, vals_hbm, out_hbm,
                  rbuf, cbuf, vbuf, idxbuf, zbuf, xsp):
    c = lax.axis_index("c")
    s = lax.axis_index("s")
    w = c * 16 + s
    pltpu.sync_copy(rows_hbm.at[w], rbuf)
    pltpu.sync_copy(cols_hbm.at[w], cbuf)
    pltpu.sync_copy(vals_hbm.at[w], vbuf)

    zero16 = jnp.zeros((16,), jnp.float32)

    def zb(i, _):
        zbuf[pl.ds(i * 16, 16)] = zero16
        return 0
    lax.fori_loop(0, 64, zb, 0)

    def zx(i, _):
        pltpu.sync_copy(zbuf, xsp.at[pl.ds(s * 81920 + i * 1024, 1024)])
        return 0
    lax.fori_loop(0, 80, zx, 0)

    def ci(i, _):
        j = i >> 3
        k = i & 7
        r = rbuf[j, pl.ds(k * 16, 16)]
        cc = cbuf[j, pl.ds(k * 16, 16)]
        idxbuf[j, pl.ds(k * 16, 16)] = r * NE + cc
        return 0
    lax.fori_loop(0, NNZ_CH * 8, ci, 0)

    plsc.subcore_barrier()

    def sc(j, _):
        pltpu.sync_copy(vbuf.at[j], xsp.at[idxbuf.at[j]], add=True)
        return 0
    lax.fori_loop(0, NNZ_CH, sc, 0)

    plsc.subcore_barrier()

    def wo(i, _):
        off = s * 81920 + i * 8192
        pltpu.sync_copy(xsp.at[pl.ds(off, 8192)], out_hbm.at[c, pl.ds(off, 8192)])
        return 0
    lax.fori_loop(0, 10, wo, 0)


_densify = functools.partial(
    pl.kernel,
    _densify_body,
    out_type=jax.ShapeDtypeStruct((2, XWORDS), jnp.float32),
    mesh=_mesh,
    compiler_params=pltpu.CompilerParams(use_tc_tiling_on_sc=False),
    scratch_types=[
        pltpu.VMEM((NNZ_CH, 128), jnp.int32),
        pltpu.VMEM((NNZ_CH, 128), jnp.int32),
        pltpu.VMEM((NNZ_CH, 128), jnp.float32),
        pltpu.VMEM((NNZ_CH, 128), jnp.int32),
        pltpu.VMEM((1024,), jnp.float32),
        pltpu.VMEM_SHARED((XWORDS,), jnp.float32),
    ],
)()


def _mm_body(x_ref, m1_ref, m2_ref, t1_ref, f_ref):
    xb = x_ref[0].reshape(2048, NE) + x_ref[1].reshape(2048, NE)
    t1_ref[...] = jnp.dot(xb, m1_ref[...], preferred_element_type=jnp.float32)
    f_ref[...] = jnp.dot(xb, m2_ref[...], preferred_element_type=jnp.float32)


def _mm(xflat, m1, m2):
    return pl.pallas_call(
        _mm_body,
        grid=(NPAD // 2048,),
        in_specs=[
            pl.BlockSpec((2, 2048 * NE), lambda i: (0, i)),
            pl.BlockSpec((NE, 80), lambda i: (0, 0)),
            pl.BlockSpec((NE, 16), lambda i: (0, 0)),
        ],
        out_specs=[
            pl.BlockSpec((2048, 80), lambda i: (i, 0)),
            pl.BlockSpec((2048, 16), lambda i: (i, 0)),
        ],
        out_shape=[
            jax.ShapeDtypeStruct((NPAD, 80), jnp.float32),
            jax.ShapeDtypeStruct((NPAD, 16), jnp.float32),
        ],
    )(xflat, m1, m2)


def _edges1_body(er_hbm, ec_hbm, f_hbm, h_hbm, acc_out,
                 erb, ecb, fb0, fb1, hb0, hb1, pay0, pay1, zb, wbuf, acc,
                 smf0, smf1, smh0, smh1, ss0, ss1, sme):
    c = lax.axis_index("c")
    s = lax.axis_index("s")
    w = c * 16 + s
    base = w * 5312
    n_tail = jnp.where(w == 31, 80, 64)

    def erow(j, _):
        pltpu.async_copy(er_hbm.at[pl.ds(base + j * 128, 128)], erb.at[j], sme)
        pltpu.async_copy(ec_hbm.at[pl.ds(base + j * 128, 128)], ecb.at[j], sme)
        return 0
    lax.fori_loop(0, 41, erow, 0)
    pltpu.async_copy(er_hbm.at[pl.ds(base + 5248, 80)], erb.at[41, pl.ds(0, 80)], sme)
    pltpu.async_copy(ec_hbm.at[pl.ds(base + 5248, 80)], ecb.at[41, pl.ds(0, 80)], sme)

    def erow_w(j, _):
        pltpu.make_async_copy(er_hbm.at[pl.ds(base + j * 128, 128)], erb.at[j], sme).wait()
        pltpu.make_async_copy(ec_hbm.at[pl.ds(base + j * 128, 128)], ecb.at[j], sme).wait()
        return 0
    lax.fori_loop(0, 41, erow_w, 0)
    pltpu.make_async_copy(er_hbm.at[pl.ds(base + 5248, 80)], erb.at[41, pl.ds(0, 80)], sme).wait()
    pltpu.make_async_copy(ec_hbm.at[pl.ds(base + 5248, 80)], ecb.at[41, pl.ds(0, 80)], sme).wait()

    iiv = lax.iota(jnp.int32, 16)
    for g in range(4, 8):
        msk = (g * 16 + iiv) < n_tail
        pad_v = 10000 + iiv + g
        erb[41, pl.ds(g * 16, 16)] = jnp.where(msk, erb[41, pl.ds(g * 16, 16)], pad_v)
        ecb[41, pl.ds(g * 16, 16)] = jnp.where(msk, ecb[41, pl.ds(g * 16, 16)], pad_v)

    zero16 = jnp.zeros((16,), jnp.float32)

    def zbody(i, _):
        r = i // 5
        k = i % 5
        zb[r, pl.ds(k * 16, 16)] = zero16
        return 0
    lax.fori_loop(0, 64 * 5, zbody, 0)

    def zacc(i, _):
        pltpu.sync_copy(zb, acc.at[pl.ds(s * 640 + i * 64, 64)])
        return 0
    lax.fori_loop(0, 10, zacc, 0)

    plsc.subcore_barrier()

    ii = lax.iota(jnp.int32, 16)
    c0 = ii >> 3
    c1 = c0 + 2
    c2 = c0 + 4
    c3 = c0 + 6

    lo = ii < 8
    c8 = (ii + 8) & 15

    def compute(fb, hb, pay):
        def wpass(i, _):
            for u in range(2):
                p = i * 2 + u
                e = p * 2
                va = fb[e, :]
                vb = fb[e + 1, :]
                vbs = jnp.take_along_axis(vb, c8, axis=0, mode="promise_in_bounds")
                flp = jnp.where(lo, va, vbs)
                fa = hb[e, pl.ds(64, 16)]
                fc = hb[e + 1, pl.ds(56, 16)]
                frp = jnp.where(lo, fa, fc)
                t = flp + frp
                lr = jnp.where(t >= 0, t, ALPHA * t)
                wbuf[pl.ds(16 * p, 16)] = jnp.exp(-lr)
            return 0
        lax.fori_loop(0, 32, wpass, 0)

        def edge4(i, _):
            for u in range(4):
                e = i * 4 + u
                wv = wbuf[pl.ds(8 * e, 16)]
                w0 = jnp.take_along_axis(wv, c0, axis=0, mode="promise_in_bounds")
                w1 = jnp.take_along_axis(wv, c1, axis=0, mode="promise_in_bounds")
                w2 = jnp.take_along_axis(wv, c2, axis=0, mode="promise_in_bounds")
                w3 = jnp.take_along_axis(wv, c3, axis=0, mode="promise_in_bounds")
                pay[e, pl.ds(0, 16)] = hb[e, pl.ds(0, 16)] * w0
                pay[e, pl.ds(16, 16)] = hb[e, pl.ds(16, 16)] * w1
                pay[e, pl.ds(32, 16)] = hb[e, pl.ds(32, 16)] * w2
                pay[e, pl.ds(48, 16)] = hb[e, pl.ds(48, 16)] * w3
                pay[e, pl.ds(64, 16)] = wv
            return 0
        lax.fori_loop(0, 32, edge4, 0)

    def gather_issue(j, fb, hb, smf, smh):
        pltpu.async_copy(f_hbm.at[erb.at[j]], fb, smf)
        pltpu.async_copy(h_hbm.at[ecb.at[j]], hb, smh)

    def gather_wait(j, fb, hb, smf, smh):
        pltpu.make_async_copy(f_hbm.at[erb.at[j]], fb, smf).wait()
        pltpu.make_async_copy(h_hbm.at[ecb.at[j]], hb, smh).wait()

    gather_issue(0, fb0, hb0, smf0, smh0)

    def pair(jj, _):
        j0 = 2 * jj
        j1 = j0 + 1
        jn = jnp.minimum(j0 + 2, E_CH - 1)
        gather_wait(j0, fb0, hb0, smf0, smh0)
        gather_issue(j1, fb1, hb1, smf1, smh1)

        @pl.when(jj > 0)
        def _():
            pltpu.make_async_copy(pay0, acc.at[erb.at[j0]], ss0).wait()
        compute(fb0, hb0, pay0)
        pltpu.async_copy(pay0, acc.at[erb.at[j0]], ss0, add=True)

        gather_wait(j1, fb1, hb1, smf1, smh1)
        gather_issue(jn, fb0, hb0, smf0, smh0)

        @pl.when(jj > 0)
        def _():
            pltpu.make_async_copy(pay1, acc.at[erb.at[j1]], ss1).wait()
        compute(fb1, hb1, pay1)
        pltpu.async_copy(pay1, acc.at[erb.at[j1]], ss1, add=True)
        return 0
    lax.fori_loop(0, E_CH // 2, pair, 0)

    gather_wait(E_CH - 1, fb0, hb0, smf0, smh0)
    pltpu.make_async_copy(pay0, acc.at[erb.at[0]], ss0).wait()
    pltpu.make_async_copy(pay1, acc.at[erb.at[0]], ss1).wait()

    plsc.subcore_barrier()

    def wo(i, _):
        r0 = s * 640 + i * 64
        pltpu.sync_copy(acc.at[pl.ds(r0, 64)], acc_out.at[c, pl.ds(r0, 64)])
        return 0
    lax.fori_loop(0, 10, wo, 0)


_edges1 = functools.partial(
    pl.kernel,
    _edges1_body,
    out_type=jax.ShapeDtypeStruct((2, NPAD, 80), jnp.float32),
    mesh=_mesh,
    compiler_params=pltpu.CompilerParams(use_tc_tiling_on_sc=False),
    scratch_types=[
        pltpu.VMEM((E_CH, 128), jnp.int32),
        pltpu.VMEM((E_CH, 128), jnp.int32),
        pltpu.VMEM((128, 16), jnp.float32),
        pltpu.VMEM((128, 16), jnp.float32),
        pltpu.VMEM((128, 80), jnp.float32),
        pltpu.VMEM((128, 80), jnp.float32),
        pltpu.VMEM((128, 80), jnp.float32),
        pltpu.VMEM((128, 80), jnp.float32),
        pltpu.VMEM((64, 80), jnp.float32),
        pltpu.VMEM((1040,), jnp.float32),
        pltpu.VMEM_SHARED((NPAD, 80), jnp.float32),
        pltpu.SemaphoreType.DMA,
        pltpu.SemaphoreType.DMA,
        pltpu.SemaphoreType.DMA,
        pltpu.SemaphoreType.DMA,
        pltpu.SemaphoreType.DMA,
        pltpu.SemaphoreType.DMA,
        pltpu.SemaphoreType.DMA,
    ],
)()


def _mid_body(a0_ref, a1_ref, k8_ref, w2_ref, t2_ref):
    acc = a0_ref[0] + a1_ref[0]
    h = acc[:, 0:64]
    rs = acc[:, 64:72]
    rse = jnp.dot(rs, k8_ref[...], preferred_element_type=jnp.float32)
    hp = h / rse
    x2 = jnp.where(hp >= 0, hp, jnp.exp(hp) - 1.0)
    t2_ref[...] = jnp.dot(x2, w2_ref[...], preferred_element_type=jnp.float32)


def _mid(acc1, k8, w2big):
    return pl.pallas_call(
        _mid_body,
        grid=(NPAD // 2048,),
        in_specs=[
            pl.BlockSpec((1, 2048, 80), lambda i: (0, i, 0)),
            pl.BlockSpec((1, 2048, 80), lambda i: (1, i, 0)),
            pl.BlockSpec((8, 64), lambda i: (0, 0)),
            pl.BlockSpec((64, 16), lambda i: (0, 0)),
        ],
        out_specs=pl.BlockSpec((2048, 16), lambda i: (i, 0)),
        out_shape=jax.ShapeDtypeStruct((NPAD, 16), jnp.float32),
    )(acc1, acc1, k8, w2big)


def _edges2_body(er_hbm, ec_hbm, t2_hbm, acc_out,
                 erb, ecb, ba0, ba1, bb0, bb1, pay0, pay1, zb, acc,
                 smf0, smf1, smh0, smh1, ss0, ss1, sme):
    c = lax.axis_index("c")
    s = lax.axis_index("s")
    w = c * 16 + s
    base = w * 5312
    n_tail = jnp.where(w == 31, 80, 64)

    def erow(j, _):
        pltpu.async_copy(er_hbm.at[pl.ds(base + j * 128, 128)], erb.at[j], sme)
        pltpu.async_copy(ec_hbm.at[pl.ds(base + j * 128, 128)], ecb.at[j], sme)
        return 0
    lax.fori_loop(0, 41, erow, 0)
    pltpu.async_copy(er_hbm.at[pl.ds(base + 5248, 80)], erb.at[41, pl.ds(0, 80)], sme)
    pltpu.async_copy(ec_hbm.at[pl.ds(base + 5248, 80)], ecb.at[41, pl.ds(0, 80)], sme)

    def erow_w(j, _):
        pltpu.make_async_copy(er_hbm.at[pl.ds(base + j * 128, 128)], erb.at[j], sme).wait()
        pltpu.make_async_copy(ec_hbm.at[pl.ds(base + j * 128, 128)], ecb.at[j], sme).wait()
        return 0
    lax.fori_loop(0, 41, erow_w, 0)
    pltpu.make_async_copy(er_hbm.at[pl.ds(base + 5248, 80)], erb.at[41, pl.ds(0, 80)], sme).wait()
    pltpu.make_async_copy(ec_hbm.at[pl.ds(base + 5248, 80)], ecb.at[41, pl.ds(0, 80)], sme).wait()

    iiv = lax.iota(jnp.int32, 16)
    for g in range(4, 8):
        msk = (g * 16 + iiv) < n_tail
        pad_v = 10000 + iiv + g
        erb[41, pl.ds(g * 16, 16)] = jnp.where(msk, erb[41, pl.ds(g * 16, 16)], pad_v)
        ecb[41, pl.ds(g * 16, 16)] = jnp.where(msk, ecb[41, pl.ds(g * 16, 16)], pad_v)

    zero16 = jnp.zeros((16,), jnp.float32)

    def zbody(i, _):
        zb[i, pl.ds(0, 16)] = zero16
        return 0
    lax.fori_loop(0, 64, zbody, 0)

    def zacc(i, _):
        pltpu.sync_copy(zb, acc.at[pl.ds(s * 640 + i * 64, 64)])
        return 0
    lax.fori_loop(0, 10, zacc, 0)

    plsc.subcore_barrier()

    ii = lax.iota(jnp.int32, 16)
    i8 = jnp.full((16,), 8, jnp.int32)
    i9 = jnp.full((16,), 9, jnp.int32)
    m8 = jnp.where(ii < 8, 1.0, 0.0)
    oh8 = jnp.where(ii == 8, 1.0, 0.0)

    def compute(ba, bb, pay):
        def edge4(i, _):
            for u in range(4):
                e = i * 4 + u
                a = ba[e, :]
                b = bb[e, :]
                t = (jnp.take_along_axis(a, i8, axis=0, mode="promise_in_bounds")
                     + jnp.take_along_axis(b, i9, axis=0, mode="promise_in_bounds"))
                lr = jnp.where(t >= 0, t, ALPHA * t)
                wv = jnp.exp(-lr)
                pay[e, :] = wv * (b * m8 + oh8)
            return 0
        lax.fori_loop(0, 32, edge4, 0)

    def gather_issue(j, ba, bb, smf, smh):
        pltpu.async_copy(t2_hbm.at[erb.at[j]], ba, smf)
        pltpu.async_copy(t2_hbm.at[ecb.at[j]], bb, smh)

    def gather_wait(j, ba, bb, smf, smh):
        pltpu.make_async_copy(t2_hbm.at[erb.at[j]], ba, smf).wait()
        pltpu.make_async_copy(t2_hbm.at[ecb.at[j]], bb, smh).wait()

    gather_issue(0, ba0, bb0, smf0, smh0)

    def pair(jj, _):
        j0 = 2 * jj
        j1 = j0 + 1
        jn = jnp.minimum(j0 + 2, E_CH - 1)
        gather_wait(j0, ba0, bb0, smf0, smh0)
        gather_issue(j1, ba1, bb1, smf1, smh1)

        @pl.when(jj > 0)
        def _():
            pltpu.make_async_copy(pay0, acc.at[erb.at[j0]], ss0).wait()
        compute(ba0, bb0, pay0)
        pltpu.async_copy(pay0, acc.at[erb.at[j0]], ss0, add=True)

        gather_wait(j1, ba1, bb1, smf1, smh1)
        gather_issue(jn, ba0, bb0, smf0, smh0)

        @pl.when(jj > 0)
        def _():
            pltpu.make_async_copy(pay1, acc.at[erb.at[j1]], ss1).wait()
        compute(ba1, bb1, pay1)
        pltpu.async_copy(pay1, acc.at[erb.at[j1]], ss1, add=True)
        return 0
    lax.fori_loop(0, E_CH // 2, pair, 0)

    gather_wait(E_CH - 1, ba0, bb0, smf0, smh0)
    pltpu.make_async_copy(pay0, acc.at[erb.at[0]], ss0).wait()
    pltpu.make_async_copy(pay1, acc.at[erb.at[0]], ss1).wait()

    plsc.subcore_barrier()

    def wo(i, _):
        r0 = s * 640 + i * 64
        pltpu.sync_copy(acc.at[pl.ds(r0, 64)], acc_out.at[c, pl.ds(r0, 64)])
        return 0
    lax.fori_loop(0, 10, wo, 0)


_edges2 = functools.partial(
    pl.kernel,
    _edges2_body,
    out_type=jax.ShapeDtypeStruct((2, NPAD, 16), jnp.float32),
    mesh=_mesh,
    compiler_params=pltpu.CompilerParams(use_tc_tiling_on_sc=False),
    scratch_types=[
        pltpu.VMEM((E_CH, 128), jnp.int32),
        pltpu.VMEM((E_CH, 128), jnp.int32),
        pltpu.VMEM((128, 16), jnp.float32),
        pltpu.VMEM((128, 16), jnp.float32),
        pltpu.VMEM((128, 16), jnp.float32),
        pltpu.VMEM((128, 16), jnp.float32),
        pltpu.VMEM((128, 16), jnp.float32),
        pltpu.VMEM((128, 16), jnp.float32),
        pltpu.VMEM((64, 16), jnp.float32),
        pltpu.VMEM_SHARED((NPAD, 16), jnp.float32),
        pltpu.SemaphoreType.DMA,
        pltpu.SemaphoreType.DMA,
        pltpu.SemaphoreType.DMA,
        pltpu.SemaphoreType.DMA,
        pltpu.SemaphoreType.DMA,
        pltpu.SemaphoreType.DMA,
        pltpu.SemaphoreType.DMA,
    ],
)()


def _fin_body(a0_ref, a1_ref, mrs_ref, o_ref):
    acc = a0_ref[0] + a1_ref[0]
    nrow = acc.shape[0]
    g = acc[:, 0:8]
    rs8 = jnp.dot(acc, mrs_ref[...], preferred_element_type=jnp.float32)
    x3 = g / rs8
    x3 = jnp.where(x3 >= 0, x3, jnp.exp(x3) - 1.0)
    li = lax.broadcasted_iota(jnp.int32, (nrow, 8), 1)
    l = jnp.where(li < Q, x3, -1e30)
    m = jnp.max(l, axis=1, keepdims=True)
    se = jnp.sum(jnp.exp(l - m), axis=1, keepdims=True)
    o_ref[...] = (l - (jnp.log(se) + m))[:, :Q]


def _fin(acc2, mrs):
    return pl.pallas_call(
        _fin_body,
        grid=(NPAD // 2048,),
        in_specs=[
            pl.BlockSpec((1, 2048, 16), lambda i: (0, i, 0)),
            pl.BlockSpec((1, 2048, 16), lambda i: (1, i, 0)),
            pl.BlockSpec((16, 8), lambda i: (0, 0)),
        ],
        out_specs=pl.BlockSpec((2048, Q), lambda i: (i, 0)),
        out_shape=jax.ShapeDtypeStruct((N, Q), jnp.float32),
    )(acc2, acc2, mrs)


def kernel(feature_indices, feature_values, edge_rows, edge_cols, W_heads, a_heads, W_out, a_out):
    npadz = NNZ_PAD - NNZ
    pr = (jnp.arange(npadz, dtype=jnp.int32) % N)
    pc = (jnp.arange(npadz, dtype=jnp.int32) % NE)
    rows_p = jnp.concatenate([feature_indices[0].astype(jnp.int32), pr]).reshape(NW, NNZ_CH, 128)
    cols_p = jnp.concatenate([feature_indices[1].astype(jnp.int32), pc]).reshape(NW, NNZ_CH, 128)
    vals_p = jnp.concatenate([feature_values, jnp.zeros((npadz,), jnp.float32)]).reshape(NW, NNZ_CH, 128)

    er_p = edge_rows.astype(jnp.int32)
    ec_p = edge_cols.astype(jnp.int32)

    W_cat = jnp.transpose(W_heads, (1, 0, 2)).reshape(NE, NHEADS * HID)
    al = a_heads[:, 0, :HID]
    ar = a_heads[:, 0, HID:]
    eye8 = jnp.eye(NHEADS, dtype=jnp.float32)
    A_l = (al[:, :, None] * eye8[:, None, :]).reshape(NHEADS * HID, NHEADS)
    A_r = (ar[:, :, None] * eye8[:, None, :]).reshape(NHEADS * HID, NHEADS)
    M1 = jnp.concatenate([W_cat, W_cat @ A_r, jnp.zeros((NE, 8), jnp.float32)], axis=1)
    M2 = jnp.concatenate([W_cat @ A_l, W_cat @ A_r], axis=1)
    K8 = jnp.repeat(eye8, HID, axis=1)
    W_out8 = jnp.pad(W_out, ((0, 0), (0, 1)))
    a2l8 = jnp.pad(a_out[0, :Q], (0, 1))
    a2r8 = jnp.pad(a_out[0, Q:2 * Q], (0, 1))
    W2big = jnp.concatenate(
        [W_out8, (W_out8 @ a2l8)[:, None], (W_out8 @ a2r8)[:, None],
         jnp.zeros((NHEADS * HID, 6), jnp.float32)], axis=1)
    mrs = jnp.zeros((16, 8), jnp.float32).at[8, :].set(1.0)

    xflat = _densify(rows_p, cols_p, vals_p)
    t1h, ftbl = _mm(xflat, M1, M2)
    acc1 = _edges1(er_p, ec_p, ftbl, t1h)
    t2 = _mid(acc1, K8, W2big)
    acc2 = _edges2(er_p, ec_p, t2)
    return _fin(acc2, mrs)


# speedup vs baseline: 1.1603x; 1.1603x over previous
import functools

import jax
import jax.numpy as jnp
from jax import lax
from jax.experimental import pallas as pl
from jax.experimental.pallas import tpu as pltpu
from jax.experimental.pallas import tpu_sc as plsc

N = 10000
NE = 128
HID = 8
NHEADS = 8
Q = 7
ALPHA = 0.2

NPAD = 10240
NW = 32
NNZ = 100000
NNZ_CH = 25
NNZ_PAD = NW * NNZ_CH * 128
E = N * 16 + N
E_CH = 42
E_PAD = NW * E_CH * 128
XWORDS = NPAD * NE

_mesh = plsc.VectorSubcoreMesh(core_axis_name="c", subcore_axis_name="s")


def _densify_body(rows_hbm, cols_hbm, vals_hbm, out_hbm,
                  rbuf, cbuf, vbuf, idxbuf, zbuf, xsp):
    c = lax.axis_index("c")
    s = lax.axis_index("s")
    w = c * 16 + s
    pltpu.sync_copy(rows_hbm.at[w], rbuf)
    pltpu.sync_copy(cols_hbm.at[w], cbuf)
    pltpu.sync_copy(vals_hbm.at[w], vbuf)

    zero16 = jnp.zeros((16,), jnp.float32)

    def zb(i, _):
        zbuf[pl.ds(i * 16, 16)] = zero16
        return 0
    lax.fori_loop(0, 64, zb, 0)

    def zx(i, _):
        pltpu.sync_copy(zbuf, xsp.at[pl.ds(s * 81920 + i * 1024, 1024)])
        return 0
    lax.fori_loop(0, 80, zx, 0)

    def ci(i, _):
        j = i >> 3
        k = i & 7
        r = rbuf[j, pl.ds(k * 16, 16)]
        cc = cbuf[j, pl.ds(k * 16, 16)]
        idxbuf[j, pl.ds(k * 16, 16)] = r * NE + cc
        return 0
    lax.fori_loop(0, NNZ_CH * 8, ci, 0)

    plsc.subcore_barrier()

    def sc(j, _):
        pltpu.sync_copy(vbuf.at[j], xsp.at[idxbuf.at[j]], add=True)
        return 0
    lax.fori_loop(0, NNZ_CH, sc, 0)

    plsc.subcore_barrier()

    def wo(i, _):
        off = s * 81920 + i * 8192
        pltpu.sync_copy(xsp.at[pl.ds(off, 8192)], out_hbm.at[c, pl.ds(off, 8192)])
        return 0
    lax.fori_loop(0, 10, wo, 0)


_densify = functools.partial(
    pl.kernel,
    _densify_body,
    out_type=jax.ShapeDtypeStruct((2, XWORDS), jnp.float32),
    mesh=_mesh,
    compiler_params=pltpu.CompilerParams(use_tc_tiling_on_sc=False),
    scratch_types=[
        pltpu.VMEM((NNZ_CH, 128), jnp.int32),
        pltpu.VMEM((NNZ_CH, 128), jnp.int32),
        pltpu.VMEM((NNZ_CH, 128), jnp.float32),
        pltpu.VMEM((NNZ_CH, 128), jnp.int32),
        pltpu.VMEM((1024,), jnp.float32),
        pltpu.VMEM_SHARED((XWORDS,), jnp.float32),
    ],
)()


def _mm_body(x_ref, m1_ref, m2_ref, t1_ref, f_ref):
    xb = x_ref[0].reshape(2048, NE) + x_ref[1].reshape(2048, NE)
    t1_ref[...] = jnp.dot(xb, m1_ref[...], preferred_element_type=jnp.float32)
    f_ref[...] = jnp.dot(xb, m2_ref[...], preferred_element_type=jnp.float32)


def _mm(xflat, m1, m2):
    return pl.pallas_call(
        _mm_body,
        grid=(NPAD // 2048,),
        in_specs=[
            pl.BlockSpec((2, 2048 * NE), lambda i: (0, i)),
            pl.BlockSpec((NE, 80), lambda i: (0, 0)),
            pl.BlockSpec((NE, 16), lambda i: (0, 0)),
        ],
        out_specs=[
            pl.BlockSpec((2048, 80), lambda i: (i, 0)),
            pl.BlockSpec((2048, 16), lambda i: (i, 0)),
        ],
        out_shape=[
            jax.ShapeDtypeStruct((NPAD, 80), jnp.float32),
            jax.ShapeDtypeStruct((NPAD, 16), jnp.float32),
        ],
    )(xflat, m1, m2)


def _edges1_body(er_hbm, ec_hbm, f_hbm, h_hbm, acc_out,
                 erb, ecb, fb0, fb1, hb0, hb1, pay0, pay1, zb, wbuf, acc, fsp,
                 smf0, smf1, smh0, smh1, ss0, ss1):
    c = lax.axis_index("c")
    s = lax.axis_index("s")
    w = c * 16 + s
    pltpu.sync_copy(er_hbm.at[w], erb)
    pltpu.sync_copy(ec_hbm.at[w], ecb)

    zero16 = jnp.zeros((16,), jnp.float32)

    def zbody(i, _):
        r = i // 5
        k = i % 5
        zb[r, pl.ds(k * 16, 16)] = zero16
        return 0
    lax.fori_loop(0, 64 * 5, zbody, 0)

    def zacc(i, _):
        pltpu.sync_copy(zb, acc.at[pl.ds(s * 640 + i * 64, 64)])
        return 0
    lax.fori_loop(0, 10, zacc, 0)

    pltpu.sync_copy(f_hbm.at[pl.ds(s * 640, 640)], fsp.at[pl.ds(s * 640, 640)])

    plsc.subcore_barrier()

    ii = lax.iota(jnp.int32, 16)
    c0 = ii >> 3
    c1 = c0 + 2
    c2 = c0 + 4
    c3 = c0 + 6

    lo = ii < 8
    c8 = (ii + 8) & 15

    def compute(fb, hb, pay):
        def wpass(i, _):
            for u in range(2):
                p = i * 2 + u
                e = p * 2
                va = fb[e, :]
                vb = fb[e + 1, :]
                vbs = jnp.take_along_axis(vb, c8, axis=0, mode="promise_in_bounds")
                flp = jnp.where(lo, va, vbs)
                fa = hb[e, pl.ds(64, 16)]
                fc = hb[e + 1, pl.ds(56, 16)]
                frp = jnp.where(lo, fa, fc)
                t = flp + frp
                lr = jnp.where(t >= 0, t, ALPHA * t)
                wbuf[pl.ds(16 * p, 16)] = jnp.exp(-lr)
            return 0
        lax.fori_loop(0, 32, wpass, 0)

        def edge4(i, _):
            for u in range(4):
                e = i * 4 + u
                wv = wbuf[pl.ds(8 * e, 16)]
                w0 = jnp.take_along_axis(wv, c0, axis=0, mode="promise_in_bounds")
                w1 = jnp.take_along_axis(wv, c1, axis=0, mode="promise_in_bounds")
                w2 = jnp.take_along_axis(wv, c2, axis=0, mode="promise_in_bounds")
                w3 = jnp.take_along_axis(wv, c3, axis=0, mode="promise_in_bounds")
                pay[e, pl.ds(0, 16)] = hb[e, pl.ds(0, 16)] * w0
                pay[e, pl.ds(16, 16)] = hb[e, pl.ds(16, 16)] * w1
                pay[e, pl.ds(32, 16)] = hb[e, pl.ds(32, 16)] * w2
                pay[e, pl.ds(48, 16)] = hb[e, pl.ds(48, 16)] * w3
                pay[e, pl.ds(64, 16)] = wv
            return 0
        lax.fori_loop(0, 32, edge4, 0)

    def gather_issue(j, fb, hb, smf, smh):
        pltpu.async_copy(fsp.at[erb.at[j]], fb, smf)
        pltpu.async_copy(h_hbm.at[ecb.at[j]], hb, smh)

    def gather_wait(j, fb, hb, smf, smh):
        pltpu.make_async_copy(fsp.at[erb.at[j]], fb, smf).wait()
        pltpu.make_async_copy(h_hbm.at[ecb.at[j]], hb, smh).wait()

    gather_issue(0, fb0, hb0, smf0, smh0)

    def pair(jj, _):
        j0 = 2 * jj
        j1 = j0 + 1
        jn = jnp.minimum(j0 + 2, E_CH - 1)
        gather_wait(j0, fb0, hb0, smf0, smh0)
        gather_issue(j1, fb1, hb1, smf1, smh1)

        @pl.when(jj > 0)
        def _():
            pltpu.make_async_copy(pay0, acc.at[erb.at[j0]], ss0).wait()
        compute(fb0, hb0, pay0)
        pltpu.async_copy(pay0, acc.at[erb.at[j0]], ss0, add=True)

        gather_wait(j1, fb1, hb1, smf1, smh1)
        gather_issue(jn, fb0, hb0, smf0, smh0)

        @pl.when(jj > 0)
        def _():
            pltpu.make_async_copy(pay1, acc.at[erb.at[j1]], ss1).wait()
        compute(fb1, hb1, pay1)
        pltpu.async_copy(pay1, acc.at[erb.at[j1]], ss1, add=True)
        return 0
    lax.fori_loop(0, E_CH // 2, pair, 0)

    gather_wait(E_CH - 1, fb0, hb0, smf0, smh0)
    pltpu.make_async_copy(pay0, acc.at[erb.at[0]], ss0).wait()
    pltpu.make_async_copy(pay1, acc.at[erb.at[0]], ss1).wait()

    plsc.subcore_barrier()

    def wo(i, _):
        r0 = s * 640 + i * 64
        pltpu.sync_copy(acc.at[pl.ds(r0, 64)], acc_out.at[c, pl.ds(r0, 64)])
        return 0
    lax.fori_loop(0, 10, wo, 0)


_edges1 = functools.partial(
    pl.kernel,
    _edges1_body,
    out_type=jax.ShapeDtypeStruct((2, NPAD, 80), jnp.float32),
    mesh=_mesh,
    compiler_params=pltpu.CompilerParams(use_tc_tiling_on_sc=False),
    scratch_types=[
        pltpu.VMEM((E_CH, 128), jnp.int32),
        pltpu.VMEM((E_CH, 128), jnp.int32),
        pltpu.VMEM((128, 16), jnp.float32),
        pltpu.VMEM((128, 16), jnp.float32),
        pltpu.VMEM((128, 80), jnp.float32),
        pltpu.VMEM((128, 80), jnp.float32),
        pltpu.VMEM((128, 80), jnp.float32),
        pltpu.VMEM((128, 80), jnp.float32),
        pltpu.VMEM((64, 80), jnp.float32),
        pltpu.VMEM((1040,), jnp.float32),
        pltpu.VMEM_SHARED((NPAD, 80), jnp.float32),
        pltpu.VMEM_SHARED((NPAD, 16), jnp.float32),
        pltpu.SemaphoreType.DMA,
        pltpu.SemaphoreType.DMA,
        pltpu.SemaphoreType.DMA,
        pltpu.SemaphoreType.DMA,
        pltpu.SemaphoreType.DMA,
        pltpu.SemaphoreType.DMA,
    ],
)()


def _mid_body(a0_ref, a1_ref, k8_ref, w2_ref, t2_ref):
    acc = a0_ref[0] + a1_ref[0]
    h = acc[:, 0:64]
    rs = acc[:, 64:72]
    rse = jnp.dot(rs, k8_ref[...], preferred_element_type=jnp.float32)
    hp = h / rse
    x2 = jnp.where(hp >= 0, hp, jnp.exp(hp) - 1.0)
    t2_ref[...] = jnp.dot(x2, w2_ref[...], preferred_element_type=jnp.float32)


def _mid(acc1, k8, w2big):
    return pl.pallas_call(
        _mid_body,
        grid=(NPAD // 2048,),
        in_specs=[
            pl.BlockSpec((1, 2048, 80), lambda i: (0, i, 0)),
            pl.BlockSpec((1, 2048, 80), lambda i: (1, i, 0)),
            pl.BlockSpec((8, 64), lambda i: (0, 0)),
            pl.BlockSpec((64, 16), lambda i: (0, 0)),
        ],
        out_specs=pl.BlockSpec((2048, 16), lambda i: (i, 0)),
        out_shape=jax.ShapeDtypeStruct((NPAD, 16), jnp.float32),
    )(acc1, acc1, k8, w2big)


def _edges2_body(er_hbm, ec_hbm, t2_hbm, acc_out,
                 erb, ecb, ba0, ba1, bb0, bb1, pay0, pay1, zb, acc, tsp,
                 smf0, smf1, smh0, smh1, ss0, ss1):
    c = lax.axis_index("c")
    s = lax.axis_index("s")
    w = c * 16 + s
    pltpu.sync_copy(er_hbm.at[w], erb)
    pltpu.sync_copy(ec_hbm.at[w], ecb)

    zero16 = jnp.zeros((16,), jnp.float32)

    def zbody(i, _):
        zb[i, pl.ds(0, 16)] = zero16
        return 0
    lax.fori_loop(0, 64, zbody, 0)

    def zacc(i, _):
        pltpu.sync_copy(zb, acc.at[pl.ds(s * 640 + i * 64, 64)])
        return 0
    lax.fori_loop(0, 10, zacc, 0)

    pltpu.sync_copy(t2_hbm.at[pl.ds(s * 640, 640)], tsp.at[pl.ds(s * 640, 640)])

    plsc.subcore_barrier()

    ii = lax.iota(jnp.int32, 16)
    i8 = jnp.full((16,), 8, jnp.int32)
    i9 = jnp.full((16,), 9, jnp.int32)
    m8 = jnp.where(ii < 8, 1.0, 0.0)
    oh8 = jnp.where(ii == 8, 1.0, 0.0)

    def compute(ba, bb, pay):
        def edge4(i, _):
            for u in range(4):
                e = i * 4 + u
                a = ba[e, :]
                b = bb[e, :]
                t = (jnp.take_along_axis(a, i8, axis=0, mode="promise_in_bounds")
                     + jnp.take_along_axis(b, i9, axis=0, mode="promise_in_bounds"))
                lr = jnp.where(t >= 0, t, ALPHA * t)
                wv = jnp.exp(-lr)
                pay[e, :] = wv * (b * m8 + oh8)
            return 0
        lax.fori_loop(0, 32, edge4, 0)

    def gather_issue(j, ba, bb, smf, smh):
        pltpu.async_copy(tsp.at[erb.at[j]], ba, smf)
        pltpu.async_copy(tsp.at[ecb.at[j]], bb, smh)

    def gather_wait(j, ba, bb, smf, smh):
        pltpu.make_async_copy(tsp.at[erb.at[j]], ba, smf).wait()
        pltpu.make_async_copy(tsp.at[ecb.at[j]], bb, smh).wait()

    gather_issue(0, ba0, bb0, smf0, smh0)

    def pair(jj, _):
        j0 = 2 * jj
        j1 = j0 + 1
        jn = jnp.minimum(j0 + 2, E_CH - 1)
        gather_wait(j0, ba0, bb0, smf0, smh0)
        gather_issue(j1, ba1, bb1, smf1, smh1)

        @pl.when(jj > 0)
        def _():
            pltpu.make_async_copy(pay0, acc.at[erb.at[j0]], ss0).wait()
        compute(ba0, bb0, pay0)
        pltpu.async_copy(pay0, acc.at[erb.at[j0]], ss0, add=True)

        gather_wait(j1, ba1, bb1, smf1, smh1)
        gather_issue(jn, ba0, bb0, smf0, smh0)

        @pl.when(jj > 0)
        def _():
            pltpu.make_async_copy(pay1, acc.at[erb.at[j1]], ss1).wait()
        compute(ba1, bb1, pay1)
        pltpu.async_copy(pay1, acc.at[erb.at[j1]], ss1, add=True)
        return 0
    lax.fori_loop(0, E_CH // 2, pair, 0)

    gather_wait(E_CH - 1, ba0, bb0, smf0, smh0)
    pltpu.make_async_copy(pay0, acc.at[erb.at[0]], ss0).wait()
    pltpu.make_async_copy(pay1, acc.at[erb.at[0]], ss1).wait()

    plsc.subcore_barrier()

    def wo(i, _):
        r0 = s * 640 + i * 64
        pltpu.sync_copy(acc.at[pl.ds(r0, 64)], acc_out.at[c, pl.ds(r0, 64)])
        return 0
    lax.fori_loop(0, 10, wo, 0)


_edges2 = functools.partial(
    pl.kernel,
    _edges2_body,
    out_type=jax.ShapeDtypeStruct((2, NPAD, 16), jnp.float32),
    mesh=_mesh,
    compiler_params=pltpu.CompilerParams(use_tc_tiling_on_sc=False),
    scratch_types=[
        pltpu.VMEM((E_CH, 128), jnp.int32),
        pltpu.VMEM((E_CH, 128), jnp.int32),
        pltpu.VMEM((128, 16), jnp.float32),
        pltpu.VMEM((128, 16), jnp.float32),
        pltpu.VMEM((128, 16), jnp.float32),
        pltpu.VMEM((128, 16), jnp.float32),
        pltpu.VMEM((128, 16), jnp.float32),
        pltpu.VMEM((128, 16), jnp.float32),
        pltpu.VMEM((64, 16), jnp.float32),
        pltpu.VMEM_SHARED((NPAD, 16), jnp.float32),
        pltpu.VMEM_SHARED((NPAD, 16), jnp.float32),
        pltpu.SemaphoreType.DMA,
        pltpu.SemaphoreType.DMA,
        pltpu.SemaphoreType.DMA,
        pltpu.SemaphoreType.DMA,
        pltpu.SemaphoreType.DMA,
        pltpu.SemaphoreType.DMA,
    ],
)()


def _fin_body(a0_ref, a1_ref, mrs_ref, o_ref):
    acc = a0_ref[0] + a1_ref[0]
    nrow = acc.shape[0]
    g = acc[:, 0:8]
    rs8 = jnp.dot(acc, mrs_ref[...], preferred_element_type=jnp.float32)
    x3 = g / rs8
    x3 = jnp.where(x3 >= 0, x3, jnp.exp(x3) - 1.0)
    li = lax.broadcasted_iota(jnp.int32, (nrow, 8), 1)
    l = jnp.where(li < Q, x3, -1e30)
    m = jnp.max(l, axis=1, keepdims=True)
    se = jnp.sum(jnp.exp(l - m), axis=1, keepdims=True)
    o_ref[...] = (l - (jnp.log(se) + m))[:, :Q]


def _fin(acc2, mrs):
    return pl.pallas_call(
        _fin_body,
        grid=(NPAD // 2048,),
        in_specs=[
            pl.BlockSpec((1, 2048, 16), lambda i: (0, i, 0)),
            pl.BlockSpec((1, 2048, 16), lambda i: (1, i, 0)),
            pl.BlockSpec((16, 8), lambda i: (0, 0)),
        ],
        out_specs=pl.BlockSpec((2048, Q), lambda i: (i, 0)),
        out_shape=jax.ShapeDtypeStruct((N, Q), jnp.float32),
    )(acc2, acc2, mrs)


def kernel(feature_indices, feature_values, edge_rows, edge_cols, W_heads, a_heads, W_out, a_out):
    npadz = NNZ_PAD - NNZ
    pr = (jnp.arange(npadz, dtype=jnp.int32) % N)
    pc = (jnp.arange(npadz, dtype=jnp.int32) % NE)
    rows_p = jnp.concatenate([feature_indices[0].astype(jnp.int32), pr]).reshape(NW, NNZ_CH, 128)
    cols_p = jnp.concatenate([feature_indices[1].astype(jnp.int32), pc]).reshape(NW, NNZ_CH, 128)
    vals_p = jnp.concatenate([feature_values, jnp.zeros((npadz,), jnp.float32)]).reshape(NW, NNZ_CH, 128)

    epadz = E_PAD - E
    pe = (jnp.arange(epadz, dtype=jnp.int32) % 32) + N
    er_p = jnp.concatenate([edge_rows.astype(jnp.int32), pe]).reshape(NW, E_CH, 128)
    ec_p = jnp.concatenate([edge_cols.astype(jnp.int32), pe]).reshape(NW, E_CH, 128)

    W_cat = jnp.transpose(W_heads, (1, 0, 2)).reshape(NE, NHEADS * HID)
    al = a_heads[:, 0, :HID]
    ar = a_heads[:, 0, HID:]
    eye8 = jnp.eye(NHEADS, dtype=jnp.float32)
    A_l = (al[:, :, None] * eye8[:, None, :]).reshape(NHEADS * HID, NHEADS)
    A_r = (ar[:, :, None] * eye8[:, None, :]).reshape(NHEADS * HID, NHEADS)
    M1 = jnp.concatenate([W_cat, W_cat @ A_r, jnp.zeros((NE, 8), jnp.float32)], axis=1)
    M2 = jnp.concatenate([W_cat @ A_l, W_cat @ A_r], axis=1)
    K8 = jnp.repeat(eye8, HID, axis=1)
    W_out8 = jnp.pad(W_out, ((0, 0), (0, 1)))
    a2l8 = jnp.pad(a_out[0, :Q], (0, 1))
    a2r8 = jnp.pad(a_out[0, Q:2 * Q], (0, 1))
    W2big = jnp.concatenate(
        [W_out8, (W_out8 @ a2l8)[:, None], (W_out8 @ a2r8)[:, None],
         jnp.zeros((NHEADS * HID, 6), jnp.float32)], axis=1)
    mrs = jnp.zeros((16, 8), jnp.float32).at[8, :].set(1.0)

    xflat = _densify(rows_p, cols_p, vals_p)
    t1h, ftbl = _mm(xflat, M1, M2)
    acc1 = _edges1(er_p, ec_p, ftbl, t1h)
    t2 = _mid(acc1, K8, W2big)
    acc2 = _edges2(er_p, ec_p, t2)
    return _fin(acc2, mrs)
